# Initial kernel scaffold; baseline (speedup 1.0000x reference)
import jax
import jax.numpy as jnp
from jax.experimental import pallas as pl


def kernel(x, edge_index, edge_pred, W1, b1, W2, b2):
    raise NotImplementedError("write your pallas kernel here")


# speedup vs baseline: 5.8858x; 5.8858x over previous
import functools

import jax
import jax.numpy as jnp
from jax import lax
from jax.experimental import pallas as pl
from jax.experimental.pallas import tpu as pltpu, tpu_sc as plsc

N = 10000
E = 320000
D = 128
H = 32
C = 30
REG = 0.01

NC = 2
NS = 16
NW = NC * NS
K = 128
CH = 79
EPW = CH * K
EP = NW * EPW
RPS = 632
RP = NS * RPS

_MESH = plsc.VectorSubcoreMesh(core_axis_name="c", subcore_axis_name="s")


@functools.partial(
    pl.kernel,
    out_type=jax.ShapeDtypeStruct((NC, RP, H), jnp.float32),
    mesh=_MESH,
    compiler_params=pltpu.CompilerParams(use_tc_tiling_on_sc=False),
    scratch_types=[
        pltpu.VMEM((CH, K), jnp.int32),
        pltpu.VMEM((CH, K), jnp.int32),
        pltpu.VMEM((K, H), jnp.float32),
        pltpu.VMEM_SHARED((RP, H), jnp.float32),
        pltpu.SemaphoreType.DMA,
    ],
)
def _sc_agg(hs_hbm, src_hbm, dst_hbm, zeros_hbm, out_hbm,
            srcv, dstv, rows, acc, sem):
    c = lax.axis_index("c")
    s = lax.axis_index("s")
    wid = s * NC + c
    row0 = pl.multiple_of(s * RPS, 8)
    pltpu.sync_copy(zeros_hbm.at[pl.ds(row0, RPS)], acc.at[pl.ds(row0, RPS)])
    plsc.subcore_barrier()
    pltpu.sync_copy(src_hbm.at[wid], srcv)
    pltpu.sync_copy(dst_hbm.at[wid], dstv)

    def body(ci, carry):
        pltpu.async_copy(hs_hbm.at[srcv.at[ci]], rows, sem).wait()
        pltpu.sync_copy(rows, acc.at[dstv.at[ci]], add=True)
        return carry

    lax.fori_loop(0, CH, body, 0)
    plsc.subcore_barrier()
    pltpu.sync_copy(acc.at[pl.ds(row0, RPS)], out_hbm.at[c, pl.ds(row0, RPS)])


def _edge_split(idx):
    pad = jnp.full((EP - E,), N, jnp.int32)
    return jnp.concatenate([idx, pad]).reshape(NW, CH, K)


def _aggregate(hs, src3, dst3, zeros_rp):
    hs_pad = jnp.zeros((RP, H), jnp.float32).at[:N].set(hs)
    parts = _sc_agg(hs_pad, src3, dst3, zeros_rp)
    return parts[0, :N] + parts[1, :N]


def kernel(x, edge_index, edge_pred, W1, b1, W2, b2):
    src, dst = edge_index[0], edge_index[1]
    src3 = _edge_split(src)
    dst3 = _edge_split(dst)
    zeros_rp = jnp.zeros((RP, H), jnp.float32)

    ones_tab = jnp.zeros((N, H), jnp.float32).at[:, 0].set(1.0)
    deg = _aggregate(ones_tab, src3, dst3, zeros_rp)[:, 0] + 1.0
    dinv = lax.rsqrt(deg)

    h1 = x @ W1
    agg1 = _aggregate(dinv[:, None] * h1, src3, dst3, zeros_rp)
    z1 = jax.nn.relu(dinv[:, None] * agg1 + dinv[:, None] ** 2 * h1 + b1)

    W2p = jnp.zeros((H, H), jnp.float32).at[:, :C].set(W2)
    h2 = z1 @ W2p
    agg2 = _aggregate(dinv[:, None] * h2, src3, dst3, zeros_rp)
    o2 = (dinv[:, None] * agg2 + dinv[:, None] ** 2 * h2)[:, :C] + b2

    FX = jax.nn.softmax(o2, axis=-1)
    FF = jnp.sum(FX[src] * FX[dst], axis=-1)
    NFX = jnp.log(1.0 - FX ** 2)
    preg = -jnp.sum(jnp.log(1.0001 - jnp.exp(jnp.sum(NFX, axis=0))), axis=0)
    loss = jnp.mean((FF - edge_pred) ** 2) + REG * preg
    return (FX, loss)


# speedup vs baseline: 12.1379x; 2.0622x over previous
import functools

import jax
import jax.numpy as jnp
from jax import lax
from jax.experimental import pallas as pl
from jax.experimental.pallas import tpu as pltpu, tpu_sc as plsc

N = 10000
E = 320000
D = 128
H = 32
C = 30
REG = 0.01

NC = 2
NS = 16
NW = NC * NS
K = 128
CH = 79
EPW = CH * K
EP = NW * EPW
RPS = 632
RP = NS * RPS

_MESH = plsc.VectorSubcoreMesh(core_axis_name="c", subcore_axis_name="s")


@functools.partial(
    pl.kernel,
    out_type=jax.ShapeDtypeStruct((NC, RP, H), jnp.float32),
    mesh=_MESH,
    compiler_params=pltpu.CompilerParams(use_tc_tiling_on_sc=False),
    scratch_types=[
        pltpu.VMEM((CH, K), jnp.int32),
        pltpu.VMEM((CH, K), jnp.int32),
        pltpu.VMEM((K, H), jnp.float32),
        pltpu.VMEM_SHARED((RP, H), jnp.float32),
        pltpu.SemaphoreType.DMA,
    ],
)
def _sc_agg(hs_hbm, src_hbm, dst_hbm, zeros_hbm, out_hbm,
            srcv, dstv, rows, acc, sem):
    c = lax.axis_index("c")
    s = lax.axis_index("s")
    wid = s * NC + c
    row0 = pl.multiple_of(s * RPS, 8)
    pltpu.sync_copy(zeros_hbm.at[pl.ds(row0, RPS)], acc.at[pl.ds(row0, RPS)])
    plsc.subcore_barrier()
    pltpu.sync_copy(src_hbm.at[wid], srcv)
    pltpu.sync_copy(dst_hbm.at[wid], dstv)

    def body(ci, carry):
        pltpu.async_copy(hs_hbm.at[srcv.at[ci]], rows, sem).wait()
        pltpu.sync_copy(rows, acc.at[dstv.at[ci]], add=True)
        return carry

    lax.fori_loop(0, CH, body, 0)
    plsc.subcore_barrier()
    pltpu.sync_copy(acc.at[pl.ds(row0, RPS)], out_hbm.at[c, pl.ds(row0, RPS)])


DW = 16


@functools.partial(
    pl.kernel,
    out_type=jax.ShapeDtypeStruct((NC, RP, DW), jnp.float32),
    mesh=_MESH,
    compiler_params=pltpu.CompilerParams(use_tc_tiling_on_sc=False),
    scratch_types=[
        pltpu.VMEM((CH, K), jnp.int32),
        pltpu.VMEM((K, DW), jnp.float32),
        pltpu.VMEM_SHARED((RP, DW), jnp.float32),
    ],
)
def _sc_deg(dst_hbm, ones_hbm, zeros_hbm, out_hbm, dstv, ones, acc):
    c = lax.axis_index("c")
    s = lax.axis_index("s")
    wid = s * NC + c
    row0 = pl.multiple_of(s * RPS, 8)
    pltpu.sync_copy(zeros_hbm.at[pl.ds(row0, RPS)], acc.at[pl.ds(row0, RPS)])
    pltpu.sync_copy(ones_hbm, ones)
    plsc.subcore_barrier()
    pltpu.sync_copy(dst_hbm.at[wid], dstv)

    def body(ci, carry):
        pltpu.sync_copy(ones, acc.at[dstv.at[ci]], add=True)
        return carry

    lax.fori_loop(0, CH, body, 0)
    plsc.subcore_barrier()
    pltpu.sync_copy(acc.at[pl.ds(row0, RPS)], out_hbm.at[c, pl.ds(row0, RPS)])


@functools.partial(
    pl.kernel,
    out_type=jax.ShapeDtypeStruct((NC, NS, 16), jnp.float32),
    mesh=_MESH,
    compiler_params=pltpu.CompilerParams(
        use_tc_tiling_on_sc=False, needs_layout_passes=False),
    scratch_types=[
        pltpu.VMEM((CH, K), jnp.int32),
        pltpu.VMEM((CH, K), jnp.int32),
        pltpu.VMEM((CH * K,), jnp.float32),
        pltpu.VMEM((K, H), jnp.float32),
        pltpu.VMEM((K, H), jnp.float32),
        pltpu.VMEM((16,), jnp.float32),
        pltpu.SemaphoreType.DMA,
        pltpu.SemaphoreType.DMA,
    ],
)
def _sc_ff(fx_hbm, src_hbm, dst_hbm, pred_hbm, out_hbm,
           srcv, dstv, predv, rowss, rowsd, ssev, sem0, sem1):
    c = lax.axis_index("c")
    s = lax.axis_index("s")
    wid = s * NC + c
    pltpu.sync_copy(src_hbm.at[wid], srcv)
    pltpu.sync_copy(dst_hbm.at[wid], dstv)
    pltpu.sync_copy(pred_hbm.at[pl.ds(wid * EPW, EPW)], predv)
    lanes = lax.iota(jnp.int32, 16)

    def body(ci, sse):
        cpS = pltpu.async_copy(fx_hbm.at[srcv.at[ci]], rowss, sem0)
        cpD = pltpu.async_copy(fx_hbm.at[dstv.at[ci]], rowsd, sem1)
        cpS.wait()
        cpD.wait()
        for e0 in range(0, K, 16):
            ridx = lanes + e0
            dot = jnp.zeros((16,), jnp.float32)
            for cc in range(H):
                col = jnp.full((16,), cc, jnp.int32)
                dot += plsc.load_gather(rowss, [ridx, col]) * plsc.load_gather(rowsd, [ridx, col])
            err = dot - predv[pl.ds(ci * K + e0, 16)]
            sse += err * err
        return sse

    sse = lax.fori_loop(0, CH, body, jnp.zeros((16,), jnp.float32))
    ssev[...] = sse
    pltpu.sync_copy(ssev, out_hbm.at[c, s])


def _edge_split(idx):
    pad = jnp.full((EP - E,), N, jnp.int32)
    return jnp.concatenate([idx, pad]).reshape(NW, CH, K)


def _aggregate(hs, src3, dst3, zeros_rp):
    hs_pad = jnp.zeros((RP, H), jnp.float32).at[:N].set(hs)
    parts = _sc_agg(hs_pad, src3, dst3, zeros_rp)
    return parts[0, :N] + parts[1, :N]


def kernel(x, edge_index, edge_pred, W1, b1, W2, b2):
    src, dst = edge_index[0], edge_index[1]
    src3 = _edge_split(src)
    dst3 = _edge_split(dst)
    zeros_rp = jnp.zeros((RP, H), jnp.float32)

    degp = _sc_deg(dst3, jnp.ones((K, DW), jnp.float32), jnp.zeros((RP, DW), jnp.float32))
    deg = degp[0, :N, 0] + degp[1, :N, 0] + 1.0
    dinv = lax.rsqrt(deg)

    h1 = x @ W1
    agg1 = _aggregate(dinv[:, None] * h1, src3, dst3, zeros_rp)
    z1 = jax.nn.relu(dinv[:, None] * agg1 + dinv[:, None] ** 2 * h1 + b1)

    W2p = jnp.zeros((H, H), jnp.float32).at[:, :C].set(W2)
    h2 = z1 @ W2p
    agg2 = _aggregate(dinv[:, None] * h2, src3, dst3, zeros_rp)
    o2 = (dinv[:, None] * agg2 + dinv[:, None] ** 2 * h2)[:, :C] + b2

    FX = jax.nn.softmax(o2, axis=-1)

    FXp = jnp.zeros((RP, H), jnp.float32).at[:N, :C].set(FX)
    predp = jnp.concatenate([edge_pred, jnp.zeros((EP - E,), jnp.float32)])
    ssep = _sc_ff(FXp, src3, dst3, predp)
    sse = jnp.sum(ssep)

    NFX = jnp.log(1.0 - FX ** 2)
    preg = -jnp.sum(jnp.log(1.0001 - jnp.exp(jnp.sum(NFX, axis=0))), axis=0)
    loss = sse / E + REG * preg
    return (FX, loss)


# speedup vs baseline: 13.0998x; 1.0792x over previous
import functools

import jax
import jax.numpy as jnp
from jax import lax
from jax.experimental import pallas as pl
from jax.experimental.pallas import tpu as pltpu, tpu_sc as plsc

N = 10000
E = 320000
D = 128
H = 32
C = 30
REG = 0.01

NC = 2
NS = 16
NW = NC * NS
K = 128
CH = 80
EPW = CH * K
EP = NW * EPW
RPS = 632
RP = NS * RPS

_MESH = plsc.VectorSubcoreMesh(core_axis_name="c", subcore_axis_name="s")


@functools.partial(
    pl.kernel,
    out_type=jax.ShapeDtypeStruct((NC, RP, H), jnp.float32),
    mesh=_MESH,
    compiler_params=pltpu.CompilerParams(use_tc_tiling_on_sc=False),
    scratch_types=[
        pltpu.VMEM((CH, K), jnp.int32),
        pltpu.VMEM((CH, K), jnp.int32),
        pltpu.VMEM((K, H), jnp.float32),
        pltpu.VMEM((K, H), jnp.float32),
        pltpu.VMEM_SHARED((RP, H), jnp.float32),
        pltpu.SemaphoreType.DMA,
        pltpu.SemaphoreType.DMA,
    ],
)
def _sc_agg(hs_hbm, src_hbm, dst_hbm, zeros_hbm, out_hbm,
            srcv, dstv, rows0, rows1, acc, sem0, sem1):
    c = lax.axis_index("c")
    s = lax.axis_index("s")
    wid = s * NC + c
    row0 = pl.multiple_of(s * RPS, 8)
    pltpu.sync_copy(zeros_hbm.at[pl.ds(row0, RPS)], acc.at[pl.ds(row0, RPS)])
    plsc.subcore_barrier()
    pltpu.sync_copy(src_hbm.at[wid], srcv)
    pltpu.sync_copy(dst_hbm.at[wid], dstv)

    pltpu.async_copy(hs_hbm.at[srcv.at[0]], rows0, sem0)
    pltpu.async_copy(hs_hbm.at[srcv.at[1]], rows1, sem1)

    def body(i, carry):
        g = 2 * i
        pltpu.make_async_copy(hs_hbm.at[srcv.at[g]], rows0, sem0).wait()
        pltpu.sync_copy(rows0, acc.at[dstv.at[g]], add=True)

        @pl.when(g + 2 < CH)
        def _():
            pltpu.async_copy(hs_hbm.at[srcv.at[g + 2]], rows0, sem0)

        pltpu.make_async_copy(hs_hbm.at[srcv.at[g + 1]], rows1, sem1).wait()
        pltpu.sync_copy(rows1, acc.at[dstv.at[g + 1]], add=True)

        @pl.when(g + 3 < CH)
        def _():
            pltpu.async_copy(hs_hbm.at[srcv.at[g + 3]], rows1, sem1)

        return carry

    lax.fori_loop(0, CH // 2, body, 0)
    plsc.subcore_barrier()
    pltpu.sync_copy(acc.at[pl.ds(row0, RPS)], out_hbm.at[c, pl.ds(row0, RPS)])


DW = 16


@functools.partial(
    pl.kernel,
    out_type=jax.ShapeDtypeStruct((NC, RP, DW), jnp.float32),
    mesh=_MESH,
    compiler_params=pltpu.CompilerParams(use_tc_tiling_on_sc=False),
    scratch_types=[
        pltpu.VMEM((CH, K), jnp.int32),
        pltpu.VMEM((K, DW), jnp.float32),
        pltpu.VMEM_SHARED((RP, DW), jnp.float32),
    ],
)
def _sc_deg(dst_hbm, ones_hbm, zeros_hbm, out_hbm, dstv, ones, acc):
    c = lax.axis_index("c")
    s = lax.axis_index("s")
    wid = s * NC + c
    row0 = pl.multiple_of(s * RPS, 8)
    pltpu.sync_copy(zeros_hbm.at[pl.ds(row0, RPS)], acc.at[pl.ds(row0, RPS)])
    pltpu.sync_copy(ones_hbm, ones)
    plsc.subcore_barrier()
    pltpu.sync_copy(dst_hbm.at[wid], dstv)

    def body(ci, carry):
        pltpu.sync_copy(ones, acc.at[dstv.at[ci]], add=True)
        return carry

    lax.fori_loop(0, CH, body, 0)
    plsc.subcore_barrier()
    pltpu.sync_copy(acc.at[pl.ds(row0, RPS)], out_hbm.at[c, pl.ds(row0, RPS)])


@functools.partial(
    pl.kernel,
    out_type=jax.ShapeDtypeStruct((NC, NS, 16), jnp.float32),
    mesh=_MESH,
    compiler_params=pltpu.CompilerParams(
        use_tc_tiling_on_sc=False, needs_layout_passes=False),
    scratch_types=[
        pltpu.VMEM((CH, K), jnp.int32),
        pltpu.VMEM((CH, K), jnp.int32),
        pltpu.VMEM((CH * K,), jnp.float32),
        pltpu.VMEM((K, H), jnp.float32),
        pltpu.VMEM((K, H), jnp.float32),
        pltpu.VMEM((K, H), jnp.float32),
        pltpu.VMEM((K, H), jnp.float32),
        pltpu.VMEM((16,), jnp.float32),
        pltpu.SemaphoreType.DMA,
        pltpu.SemaphoreType.DMA,
        pltpu.SemaphoreType.DMA,
        pltpu.SemaphoreType.DMA,
    ],
)
def _sc_ff(fx_hbm, src_hbm, dst_hbm, pred_hbm, out_hbm,
           srcv, dstv, predv, rs0, rd0, rs1, rd1, ssev,
           semS0, semD0, semS1, semD1):
    c = lax.axis_index("c")
    s = lax.axis_index("s")
    wid = s * NC + c
    pltpu.sync_copy(src_hbm.at[wid], srcv)
    pltpu.sync_copy(dst_hbm.at[wid], dstv)
    pltpu.sync_copy(pred_hbm.at[pl.ds(wid * EPW, EPW)], predv)
    lanes = lax.iota(jnp.int32, 16)

    def start(ci, rows_s, rows_d, sem_s, sem_d):
        pltpu.async_copy(fx_hbm.at[srcv.at[ci]], rows_s, sem_s)
        pltpu.async_copy(fx_hbm.at[dstv.at[ci]], rows_d, sem_d)

    def wait(ci, rows_s, rows_d, sem_s, sem_d):
        pltpu.make_async_copy(fx_hbm.at[srcv.at[ci]], rows_s, sem_s).wait()
        pltpu.make_async_copy(fx_hbm.at[dstv.at[ci]], rows_d, sem_d).wait()

    def chunk_dot(ci, rows_s, rows_d, sse):
        for e0 in range(0, K, 16):
            ridx = lanes + e0
            dot = jnp.zeros((16,), jnp.float32)
            for cc in range(H):
                col = jnp.full((16,), cc, jnp.int32)
                dot += plsc.load_gather(rows_s, [ridx, col]) * plsc.load_gather(rows_d, [ridx, col])
            err = dot - predv[pl.ds(ci * K + e0, 16)]
            sse += err * err
        return sse

    start(0, rs0, rd0, semS0, semD0)
    start(1, rs1, rd1, semS1, semD1)

    def body(i, sse):
        g = 2 * i
        wait(g, rs0, rd0, semS0, semD0)
        sse = chunk_dot(g, rs0, rd0, sse)

        @pl.when(g + 2 < CH)
        def _():
            start(g + 2, rs0, rd0, semS0, semD0)

        wait(g + 1, rs1, rd1, semS1, semD1)
        sse = chunk_dot(g + 1, rs1, rd1, sse)

        @pl.when(g + 3 < CH)
        def _():
            start(g + 3, rs1, rd1, semS1, semD1)

        return sse

    sse = lax.fori_loop(0, CH // 2, body, jnp.zeros((16,), jnp.float32))
    ssev[...] = sse
    pltpu.sync_copy(ssev, out_hbm.at[c, s])


def _edge_split(idx):
    pad = jnp.full((EP - E,), N, jnp.int32)
    return jnp.concatenate([idx, pad]).reshape(NW, CH, K)


def _aggregate(hs, src3, dst3, zeros_rp):
    hs_pad = jnp.zeros((RP, H), jnp.float32).at[:N].set(hs)
    parts = _sc_agg(hs_pad, src3, dst3, zeros_rp)
    return parts[0, :N] + parts[1, :N]


def kernel(x, edge_index, edge_pred, W1, b1, W2, b2):
    src, dst = edge_index[0], edge_index[1]
    src3 = _edge_split(src)
    dst3 = _edge_split(dst)
    zeros_rp = jnp.zeros((RP, H), jnp.float32)

    degp = _sc_deg(dst3, jnp.ones((K, DW), jnp.float32), jnp.zeros((RP, DW), jnp.float32))
    deg = degp[0, :N, 0] + degp[1, :N, 0] + 1.0
    dinv = lax.rsqrt(deg)

    h1 = x @ W1
    agg1 = _aggregate(dinv[:, None] * h1, src3, dst3, zeros_rp)
    z1 = jax.nn.relu(dinv[:, None] * agg1 + dinv[:, None] ** 2 * h1 + b1)

    W2p = jnp.zeros((H, H), jnp.float32).at[:, :C].set(W2)
    h2 = z1 @ W2p
    agg2 = _aggregate(dinv[:, None] * h2, src3, dst3, zeros_rp)
    o2 = (dinv[:, None] * agg2 + dinv[:, None] ** 2 * h2)[:, :C] + b2

    FX = jax.nn.softmax(o2, axis=-1)

    FXp = jnp.zeros((RP, H), jnp.float32).at[:N, :C].set(FX)
    predp = jnp.concatenate([edge_pred, jnp.zeros((EP - E,), jnp.float32)])
    ssep = _sc_ff(FXp, src3, dst3, predp)
    sse = jnp.sum(ssep)

    NFX = jnp.log(1.0 - FX ** 2)
    preg = -jnp.sum(jnp.log(1.0001 - jnp.exp(jnp.sum(NFX, axis=0))), axis=0)
    loss = sse / E + REG * preg
    return (FX, loss)
